# Initial kernel scaffold; baseline (speedup 1.0000x reference)
import jax
import jax.numpy as jnp
from jax.experimental import pallas as pl


def kernel(Gu, Gi, Pi, Fi, Wc0u, Wc0i, bc0, Wc1, bc1, Wi0u, Wi0iv, Wi0ip, Wi0ix, bi0, Wi1, bi1, user, item, user_pos):
    raise NotImplementedError("write your pallas kernel here")


# speedup vs baseline: 107.9340x; 107.9340x over previous
import jax
import jax.numpy as jnp
from jax.experimental import pallas as pl
from jax.experimental.pallas import tpu as pltpu
from jax.experimental.pallas import tpu_sc as plsc

B = 256
F = 64


def _sc_gathers(gu_pad, gip, user, item):
    mesh = plsc.VectorSubcoreMesh(core_axis_name="core", subcore_axis_name="subcore")
    window = 128

    @pl.kernel(
        out_type=(
            jax.ShapeDtypeStruct((B, 128), jnp.float32),
            jax.ShapeDtypeStruct((B, 128), jnp.float32),
        ),
        mesh=mesh,
    )
    def gather_kernel(gu_hbm, gip_hbm, u_hbm, i_hbm, ou_hbm, oi_hbm):
        def u_body(i_vmem, o_vmem):
            pltpu.sync_copy(gu_hbm.at[i_vmem.at[0]], o_vmem)

        def i_body(i_vmem, o_vmem):
            pltpu.sync_copy(gip_hbm.at[i_vmem.at[0]], o_vmem)

        for body, idx_hbm, out_hbm in ((u_body, u_hbm, ou_hbm), (i_body, i_hbm, oi_hbm)):
            pltpu.emit_pipeline(
                body,
                grid=(B // window,),
                in_specs=[pl.BlockSpec((1, window), index_map=lambda i: (0, i))],
                out_specs=[pl.BlockSpec((window, 128), index_map=lambda i: (i, 0))],
                core_axis_name="subcore",
                dimension_semantics=(pltpu.PARALLEL,),
            )(idx_hbm, out_hbm)

    return gather_kernel(gu_pad, gip, user.reshape(1, B), item.reshape(1, B))


def _tc_all_a(user_pos2d, Pi):
    def body(up_ref, pi_ref, out_ref):
        mask = (up_ref[...] > 0).astype(jnp.float32)
        out_ref[...] = jax.lax.dot_general(
            mask,
            pi_ref[...],
            (((1,), (0,)), ((), ())),
            precision=jax.lax.Precision.HIGHEST,
            preferred_element_type=jnp.float32,
        )

    return pl.pallas_call(
        body,
        out_shape=jax.ShapeDtypeStruct((B, F), jnp.float32),
    )(user_pos2d, Pi)


def _tc_xui(g_u, all_a, gamma_i):
    def body(gu_ref, aa_ref, gi_ref, out_ref):
        prod = (gu_ref[...] + aa_ref[...]) * gi_ref[...]
        out_ref[0, :] = jnp.sum(prod, axis=1)

    out = pl.pallas_call(
        body,
        out_shape=jax.ShapeDtypeStruct((1, B), jnp.float32),
    )(g_u, all_a, gamma_i)
    return out.reshape(B)


def kernel(Gu, Gi, Pi, Fi, Wc0u, Wc0i, bc0, Wc1, bc1, Wi0u, Wi0iv, Wi0ip, Wi0ix, bi0, Wi1, bi1, user, item, user_pos):
    user = user.astype(jnp.int32)
    item = item.astype(jnp.int32)
    user_pos2d = user_pos.reshape(B, user_pos.shape[2])

    gu_pad = jnp.pad(Gu, ((0, 0), (0, 128 - F)))
    gip = jnp.concatenate([Gi, Pi], axis=1)
    gu_rows, gip_rows = _sc_gathers(gu_pad, gip, user, item)
    g_u = gu_rows[:, :F]
    gamma_i = gip_rows[:, :F]
    p_i = gip_rows[:, F:]

    all_a = _tc_all_a(user_pos2d, Pi)
    xui = _tc_xui(g_u, all_a, gamma_i)
    return (xui, g_u, gamma_i, p_i)
